# Initial kernel scaffold; baseline (speedup 1.0000x reference)
import jax
import jax.numpy as jnp
from jax.experimental import pallas as pl


def kernel(embeddings, labels, prototypes):
    raise NotImplementedError("write your pallas kernel here")


# speedup vs baseline: 1.1763x; 1.1763x over previous
import functools

import jax
import jax.numpy as jnp
from jax import lax
from jax.experimental import pallas as pl
from jax.experimental.pallas import tpu as pltpu
from jax.experimental.pallas import tpu_sc as plsc

_EPS = 1e-12

N = 16384
D = 128
P = 1000
L = 16
NC = 2
NS = 16
NW = NC * NS
RPW = N // NW
CH = 128
NCH = RPW // CH
DSL = D // L

_sc_mesh = plsc.VectorSubcoreMesh(core_axis_name="c", subcore_axis_name="s")


@functools.partial(
    pl.kernel,
    mesh=_sc_mesh,
    out_type=[
        jax.ShapeDtypeStruct((N, L), jnp.float32),
        jax.ShapeDtypeStruct((N, L), jnp.float32),
        jax.ShapeDtypeStruct((N, L), jnp.float32),
    ],
    scratch_types=[
        pltpu.VMEM((CH,), jnp.int32),
        pltpu.VMEM((CH, D), jnp.float32),
        pltpu.VMEM((CH, D), jnp.float32),
        pltpu.VMEM((CH, L), jnp.float32),
        pltpu.VMEM((CH, L), jnp.float32),
        pltpu.VMEM((CH, L), jnp.float32),
        pltpu.SemaphoreType.DMA,
    ],
)
def _sc_dots(emb_hbm, lab_hbm, pro_hbm, dot_hbm, pp_hbm, ee_hbm,
             idx_v, emb_v, pro_v, dot_v, pp_v, ee_v, sem):
    wid = lax.axis_index("s") * NC + lax.axis_index("c")
    base = wid * RPW

    def chunk_body(k, carry):
        row0 = base + k * CH
        pltpu.sync_copy(lab_hbm.at[pl.ds(row0, CH)], idx_v)
        gat = pltpu.async_copy(pro_hbm.at[idx_v], pro_v, sem)
        pltpu.sync_copy(emb_hbm.at[pl.ds(row0, CH)], emb_v)
        gat.wait()

        def row_body(r, c):
            accd = jnp.zeros((L,), jnp.float32)
            accp = jnp.zeros((L,), jnp.float32)
            acce = jnp.zeros((L,), jnp.float32)
            for s in range(DSL):
                e = emb_v[r, pl.ds(s * L, L)]
                p = pro_v[r, pl.ds(s * L, L)]
                accd = accd + e * p
                accp = accp + p * p
                acce = acce + e * e
            dot_v[r, :] = accd
            pp_v[r, :] = accp
            ee_v[r, :] = acce
            return c

        lax.fori_loop(0, CH, row_body, 0)
        pltpu.sync_copy(dot_v, dot_hbm.at[pl.ds(row0, CH)])
        pltpu.sync_copy(pp_v, pp_hbm.at[pl.ds(row0, CH)])
        pltpu.sync_copy(ee_v, ee_hbm.at[pl.ds(row0, CH)])
        return carry

    lax.fori_loop(0, NCH, chunk_body, 0)


def _tc_finish(dot_ref, pp_ref, ee_ref, o_ref):
    dd = jnp.sum(dot_ref[...], axis=2)
    sp2 = jnp.sum(pp_ref[...], axis=2)
    se2 = jnp.sum(ee_ref[...], axis=2)
    na = jnp.maximum(jnp.sqrt(se2), _EPS)
    nb = jnp.maximum(jnp.sqrt(sp2), _EPS)
    d2 = se2 / (na * na) + sp2 / (nb * nb) - 2.0 * (dd / (na * nb))
    d = jnp.sqrt(jnp.maximum(d2, 0.0))
    o_ref[...] = jnp.sum(d, keepdims=True) / N


def kernel(embeddings, labels, prototypes):
    dot16, pp16, ee16 = _sc_dots(embeddings, labels, prototypes)
    out = pl.pallas_call(
        _tc_finish,
        out_shape=jax.ShapeDtypeStruct((1, 1), jnp.float32),
    )(
        dot16.reshape(128, 128, L),
        pp16.reshape(128, 128, L),
        ee16.reshape(128, 128, L),
    )
    return out[0, 0]


# speedup vs baseline: 1.2187x; 1.0361x over previous
import functools

import jax
import jax.numpy as jnp
from jax import lax
from jax.experimental import pallas as pl
from jax.experimental.pallas import tpu as pltpu
from jax.experimental.pallas import tpu_sc as plsc

_EPS = 1e-12

N = 16384
D = 128
P = 1000
L = 16
NC = 2
NS = 16
NW = NC * NS
RPW = N // NW
CH = 128
NCH = RPW // CH
DSL = D // L

_sc_mesh = plsc.VectorSubcoreMesh(core_axis_name="c", subcore_axis_name="s")


@functools.partial(
    pl.kernel,
    mesh=_sc_mesh,
    out_type=[
        jax.ShapeDtypeStruct((N, L), jnp.float32),
        jax.ShapeDtypeStruct((N, L), jnp.float32),
        jax.ShapeDtypeStruct((N, L), jnp.float32),
    ],
    scratch_types=[
        pltpu.VMEM((CH,), jnp.int32),
        pltpu.VMEM((CH,), jnp.int32),
        pltpu.VMEM((CH, D), jnp.float32),
        pltpu.VMEM((CH, D), jnp.float32),
        pltpu.VMEM((CH, D), jnp.float32),
        pltpu.VMEM((CH, D), jnp.float32),
        pltpu.VMEM((CH, L), jnp.float32),
        pltpu.VMEM((CH, L), jnp.float32),
        pltpu.VMEM((CH, L), jnp.float32),
        pltpu.SemaphoreType.DMA,
        pltpu.SemaphoreType.DMA,
        pltpu.SemaphoreType.DMA,
        pltpu.SemaphoreType.DMA,
    ],
)
def _sc_dots(emb_hbm, lab_hbm, pro_hbm, dot_hbm, pp_hbm, ee_hbm,
             idx0, idx1, emb0, emb1, pro0, pro1, dot_v, pp_v, ee_v,
             gsem0, gsem1, esem0, esem1):
    wid = lax.axis_index("s") * NC + lax.axis_index("c")
    base = wid * RPW
    bufs = ((idx0, emb0, pro0, gsem0, esem0),
            (idx1, emb1, pro1, gsem1, esem1))

    def _start(k):
        idx_v, emb_v, pro_v, gsem, esem = bufs[k % 2]
        row0 = base + k * CH
        pltpu.sync_copy(lab_hbm.at[pl.ds(row0, CH)], idx_v)
        g = pltpu.async_copy(pro_hbm.at[idx_v], pro_v, gsem)
        e = pltpu.async_copy(emb_hbm.at[pl.ds(row0, CH)], emb_v, esem)
        return g, e

    pend = {0: _start(0)}
    for k in range(NCH):
        if k + 1 < NCH:
            pend[k + 1] = _start(k + 1)
        g, e = pend.pop(k)
        g.wait()
        e.wait()
        _, emb_v, pro_v, _, _ = bufs[k % 2]

        def row_body(r, c, emb_v=emb_v, pro_v=pro_v):
            accd = jnp.zeros((L,), jnp.float32)
            accp = jnp.zeros((L,), jnp.float32)
            acce = jnp.zeros((L,), jnp.float32)
            for s in range(DSL):
                ev = emb_v[r, pl.ds(s * L, L)]
                pv = pro_v[r, pl.ds(s * L, L)]
                accd = accd + ev * pv
                accp = accp + pv * pv
                acce = acce + ev * ev
            dot_v[r, :] = accd
            pp_v[r, :] = accp
            ee_v[r, :] = acce
            return c

        lax.fori_loop(0, CH, row_body, 0)
        row0 = base + k * CH
        pltpu.sync_copy(dot_v, dot_hbm.at[pl.ds(row0, CH)])
        pltpu.sync_copy(pp_v, pp_hbm.at[pl.ds(row0, CH)])
        pltpu.sync_copy(ee_v, ee_hbm.at[pl.ds(row0, CH)])


def _tc_finish(dot_ref, pp_ref, ee_ref, o_ref):
    jj = lax.broadcasted_iota(jnp.int32, (D, 8), 0)
    gg = lax.broadcasted_iota(jnp.int32, (D, 8), 1)
    sel = jnp.where(jj // L == gg, 1.0, 0.0).astype(jnp.float32)
    dd = jnp.dot(dot_ref[...], sel, preferred_element_type=jnp.float32)
    sp2 = jnp.dot(pp_ref[...], sel, preferred_element_type=jnp.float32)
    se2 = jnp.dot(ee_ref[...], sel, preferred_element_type=jnp.float32)
    na = jnp.maximum(jnp.sqrt(se2), _EPS)
    nb = jnp.maximum(jnp.sqrt(sp2), _EPS)
    d2 = se2 / (na * na) + sp2 / (nb * nb) - 2.0 * (dd / (na * nb))
    d = jnp.sqrt(jnp.maximum(d2, 0.0))
    o_ref[...] = jnp.sum(d, keepdims=True) / N


def kernel(embeddings, labels, prototypes):
    dot16, pp16, ee16 = _sc_dots(embeddings, labels, prototypes)
    out = pl.pallas_call(
        _tc_finish,
        out_shape=jax.ShapeDtypeStruct((1, 1), jnp.float32),
    )(
        dot16.reshape(N * L // D, D),
        pp16.reshape(N * L // D, D),
        ee16.reshape(N * L // D, D),
    )
    return out[0, 0]


# speedup vs baseline: 1.9237x; 1.5784x over previous
import functools

import jax
import jax.numpy as jnp
from jax import lax
from jax.experimental import pallas as pl
from jax.experimental.pallas import tpu as pltpu
from jax.experimental.pallas import tpu_sc as plsc

_EPS = 1e-12

N = 16384
D = 128
P = 1000
L = 16
NC = 2
NS = 16
NW = NC * NS
RPW = N // NW
CH = 128
NCH = RPW // CH
DSL = D // L

_sc_mesh = plsc.VectorSubcoreMesh(core_axis_name="c", subcore_axis_name="s")


@functools.partial(
    pl.kernel,
    mesh=_sc_mesh,
    out_type=[
        jax.ShapeDtypeStruct((N * L,), jnp.float32),
        jax.ShapeDtypeStruct((N * L,), jnp.float32),
        jax.ShapeDtypeStruct((N * L,), jnp.float32),
    ],
    scratch_types=[
        pltpu.VMEM((CH,), jnp.int32),
        pltpu.VMEM((CH,), jnp.int32),
        pltpu.VMEM((CH, D), jnp.float32),
        pltpu.VMEM((CH, D), jnp.float32),
        pltpu.VMEM((CH, D), jnp.float32),
        pltpu.VMEM((CH, D), jnp.float32),
        pltpu.VMEM((CH * L,), jnp.float32),
        pltpu.VMEM((CH * L,), jnp.float32),
        pltpu.VMEM((CH * L,), jnp.float32),
        pltpu.SemaphoreType.DMA,
        pltpu.SemaphoreType.DMA,
        pltpu.SemaphoreType.DMA,
        pltpu.SemaphoreType.DMA,
    ],
)
def _sc_dots(emb_hbm, lab_hbm, pro_hbm, dot_hbm, pp_hbm, ee_hbm,
             idx0, idx1, emb0, emb1, pro0, pro1, dot_v, pp_v, ee_v,
             gsem0, gsem1, esem0, esem1):
    wid = lax.axis_index("s") * NC + lax.axis_index("c")
    base = wid * RPW
    bufs = ((idx0, emb0, pro0, gsem0, esem0),
            (idx1, emb1, pro1, gsem1, esem1))

    def _start(k):
        idx_v, emb_v, pro_v, gsem, esem = bufs[k % 2]
        row0 = base + k * CH
        pltpu.sync_copy(lab_hbm.at[pl.ds(row0, CH)], idx_v)
        g = pltpu.async_copy(pro_hbm.at[idx_v], pro_v, gsem)
        e = pltpu.async_copy(emb_hbm.at[pl.ds(row0, CH)], emb_v, esem)
        return g, e

    pend = {0: _start(0)}
    for k in range(NCH):
        if k + 1 < NCH:
            pend[k + 1] = _start(k + 1)
        g, e = pend.pop(k)
        g.wait()
        e.wait()
        _, emb_v, pro_v, _, _ = bufs[k % 2]

        def row_blk(rr, c, emb_v=emb_v, pro_v=pro_v):
            off = rr * D
            for j in range(D // L):
                r = rr * (D // L) + j
                accd = jnp.zeros((L,), jnp.float32)
                accp = jnp.zeros((L,), jnp.float32)
                acce = jnp.zeros((L,), jnp.float32)
                for s in range(DSL):
                    ev = emb_v[r, pl.ds(s * L, L)]
                    pv = pro_v[r, pl.ds(s * L, L)]
                    accd = accd + ev * pv
                    accp = accp + pv * pv
                    acce = acce + ev * ev
                dot_v[pl.ds(off + j * L, L)] = accd
                pp_v[pl.ds(off + j * L, L)] = accp
                ee_v[pl.ds(off + j * L, L)] = acce
            return c

        lax.fori_loop(0, CH * L // D, row_blk, 0)
        out0 = (base + k * CH) * L
        pltpu.sync_copy(dot_v, dot_hbm.at[pl.ds(out0, CH * L)])
        pltpu.sync_copy(pp_v, pp_hbm.at[pl.ds(out0, CH * L)])
        pltpu.sync_copy(ee_v, ee_hbm.at[pl.ds(out0, CH * L)])


def _tc_finish(dot_ref, pp_ref, ee_ref, o_ref):
    jj = lax.broadcasted_iota(jnp.int32, (D, 8), 0)
    gg = lax.broadcasted_iota(jnp.int32, (D, 8), 1)
    sel = jnp.where(jj // L == gg, 1.0, 0.0).astype(jnp.float32)
    dd = jnp.dot(dot_ref[...], sel, preferred_element_type=jnp.float32)
    sp2 = jnp.dot(pp_ref[...], sel, preferred_element_type=jnp.float32)
    se2 = jnp.dot(ee_ref[...], sel, preferred_element_type=jnp.float32)
    na = jnp.maximum(jnp.sqrt(se2), _EPS)
    nb = jnp.maximum(jnp.sqrt(sp2), _EPS)
    d2 = se2 / (na * na) + sp2 / (nb * nb) - 2.0 * (dd / (na * nb))
    d = jnp.sqrt(jnp.maximum(d2, 0.0))
    o_ref[...] = jnp.sum(d, keepdims=True) / N


def kernel(embeddings, labels, prototypes):
    dot16, pp16, ee16 = _sc_dots(embeddings, labels, prototypes)
    out = pl.pallas_call(
        _tc_finish,
        out_shape=jax.ShapeDtypeStruct((1, 1), jnp.float32),
    )(
        dot16.reshape(N * L // D, D),
        pp16.reshape(N * L // D, D),
        ee16.reshape(N * L // D, D),
    )
    return out[0, 0]


# speedup vs baseline: 1.9580x; 1.0178x over previous
import functools

import jax
import jax.numpy as jnp
from jax import lax
from jax.experimental import pallas as pl
from jax.experimental.pallas import tpu as pltpu
from jax.experimental.pallas import tpu_sc as plsc

_EPS = 1e-12

N = 16384
D = 128
P = 1000
L = 16
NC = 2
NS = 16
NW = NC * NS
RPW = N // NW
CH = 128
NCH = RPW // CH
DSL = D // L

_sc_mesh = plsc.VectorSubcoreMesh(core_axis_name="c", subcore_axis_name="s")


@functools.partial(
    pl.kernel,
    mesh=_sc_mesh,
    out_type=[
        jax.ShapeDtypeStruct((N * L,), jnp.float32),
        jax.ShapeDtypeStruct((N * L,), jnp.float32),
        jax.ShapeDtypeStruct((N * L,), jnp.float32),
    ],
    scratch_types=[
        pltpu.VMEM((RPW,), jnp.int32),
        pltpu.VMEM((CH, D), jnp.float32),
        pltpu.VMEM((CH, D), jnp.float32),
        pltpu.VMEM((CH, D), jnp.float32),
        pltpu.VMEM((CH, D), jnp.float32),
        pltpu.VMEM((CH * L,), jnp.float32),
        pltpu.VMEM((CH * L,), jnp.float32),
        pltpu.VMEM((CH * L,), jnp.float32),
        pltpu.VMEM((CH * L,), jnp.float32),
        pltpu.VMEM((CH * L,), jnp.float32),
        pltpu.VMEM((CH * L,), jnp.float32),
        pltpu.SemaphoreType.DMA,
        pltpu.SemaphoreType.DMA,
        pltpu.SemaphoreType.DMA,
        pltpu.SemaphoreType.DMA,
        pltpu.SemaphoreType.DMA,
        pltpu.SemaphoreType.DMA,
    ],
)
def _sc_dots(emb_hbm, lab_hbm, pro_hbm, dot_hbm, pp_hbm, ee_hbm,
             lab_v, emb0, emb1, pro0, pro1,
             dot0, dot1, pp0, pp1, ee0, ee1,
             gsem0, gsem1, esem0, esem1, osem0, osem1):
    wid = lax.axis_index("s") * NC + lax.axis_index("c")
    base = wid * RPW
    ibufs = ((emb0, pro0, gsem0, esem0), (emb1, pro1, gsem1, esem1))
    obufs = ((dot0, pp0, ee0, osem0), (dot1, pp1, ee1, osem1))

    pltpu.sync_copy(lab_hbm.at[pl.ds(base, RPW)], lab_v)

    def _start(k):
        emb_v, pro_v, gsem, esem = ibufs[k % 2]
        g = pltpu.async_copy(
            pro_hbm.at[lab_v.at[pl.ds(k * CH, CH)]], pro_v, gsem)
        e = pltpu.async_copy(
            emb_hbm.at[pl.ds(base + k * CH, CH)], emb_v, esem)
        return g, e

    pend = {0: _start(0)}
    pend_out = [None, None]
    for k in range(NCH):
        if k + 1 < NCH:
            pend[k + 1] = _start(k + 1)
        g, e = pend.pop(k)
        g.wait()
        e.wait()
        emb_v, pro_v = ibufs[k % 2][:2]
        dot_v, pp_v, ee_v, osem = obufs[k % 2]
        if pend_out[k % 2] is not None:
            for h in pend_out[k % 2]:
                h.wait()
            pend_out[k % 2] = None

        def row_blk(rr, c, emb_v=emb_v, pro_v=pro_v,
                    dot_v=dot_v, pp_v=pp_v, ee_v=ee_v):
            off = rr * D
            for j in range(D // L):
                r = rr * (D // L) + j
                accd = jnp.zeros((L,), jnp.float32)
                accp = jnp.zeros((L,), jnp.float32)
                acce = jnp.zeros((L,), jnp.float32)
                for s in range(DSL):
                    ev = emb_v[r, pl.ds(s * L, L)]
                    pv = pro_v[r, pl.ds(s * L, L)]
                    accd = accd + ev * pv
                    accp = accp + pv * pv
                    acce = acce + ev * ev
                dot_v[pl.ds(off + j * L, L)] = accd
                pp_v[pl.ds(off + j * L, L)] = accp
                ee_v[pl.ds(off + j * L, L)] = acce
            return c

        lax.fori_loop(0, CH * L // D, row_blk, 0)
        out0 = (base + k * CH) * L
        pend_out[k % 2] = (
            pltpu.async_copy(dot_v, dot_hbm.at[pl.ds(out0, CH * L)], osem),
            pltpu.async_copy(pp_v, pp_hbm.at[pl.ds(out0, CH * L)], osem),
            pltpu.async_copy(ee_v, ee_hbm.at[pl.ds(out0, CH * L)], osem),
        )
    for po in pend_out:
        if po is not None:
            for h in po:
                h.wait()


def _tc_finish(dot_ref, pp_ref, ee_ref, o_ref):
    jj = lax.broadcasted_iota(jnp.int32, (D, 8), 0)
    gg = lax.broadcasted_iota(jnp.int32, (D, 8), 1)
    sel = jnp.where(jj // L == gg, 1.0, 0.0).astype(jnp.float32)
    dd = jnp.dot(dot_ref[...], sel, preferred_element_type=jnp.float32)
    sp2 = jnp.dot(pp_ref[...], sel, preferred_element_type=jnp.float32)
    se2 = jnp.dot(ee_ref[...], sel, preferred_element_type=jnp.float32)
    na = jnp.maximum(jnp.sqrt(se2), _EPS)
    nb = jnp.maximum(jnp.sqrt(sp2), _EPS)
    d2 = se2 / (na * na) + sp2 / (nb * nb) - 2.0 * (dd / (na * nb))
    d = jnp.sqrt(jnp.maximum(d2, 0.0))
    o_ref[...] = jnp.sum(d, keepdims=True) / N


def kernel(embeddings, labels, prototypes):
    dot16, pp16, ee16 = _sc_dots(embeddings, labels, prototypes)
    out = pl.pallas_call(
        _tc_finish,
        out_shape=jax.ShapeDtypeStruct((1, 1), jnp.float32),
    )(
        dot16.reshape(N * L // D, D),
        pp16.reshape(N * L // D, D),
        ee16.reshape(N * L // D, D),
    )
    return out[0, 0]
